# Initial kernel scaffold; baseline (speedup 1.0000x reference)
import jax
import jax.numpy as jnp
from jax.experimental import pallas as pl


def kernel(features, labels, proto_0, proto_1):
    raise NotImplementedError("write your pallas kernel here")


# speedup vs baseline: 2.0175x; 2.0175x over previous
import functools

import jax
import jax.numpy as jnp
from jax import lax
from jax.experimental import pallas as pl
from jax.experimental.pallas import tpu as pltpu
from jax.experimental.pallas import tpu_sc as plsc

LAMBDA = 1.0

_NC = 2
_NS = 16
_NW = _NC * _NS
_L = 16

_ROWS = 16384
_D = 128
_RPW = _ROWS // _NW
_CHUNKS = _D // _L


def _make_sc_partials():
    mesh = plsc.VectorSubcoreMesh(core_axis_name="c", subcore_axis_name="s")

    @functools.partial(
        pl.kernel,
        mesh=mesh,
        out_type=jax.ShapeDtypeStruct((_NW, _L), jnp.float32),
        scratch_types=[
            pltpu.VMEM((_RPW, _D), jnp.float32),
            pltpu.VMEM((_RPW,), jnp.int32),
            pltpu.VMEM((1, _D), jnp.float32),
            pltpu.VMEM((1, _D), jnp.float32),
            pltpu.VMEM((_L,), jnp.float32),
        ],
    )
    def sc_partials(feat_hbm, lab_hbm, c0_hbm, c1_hbm, out_hbm,
                    feat_v, lab_v, c0_v, c1_v, acc_v):
        wid = lax.axis_index("s") * _NC + lax.axis_index("c")
        base = wid * _RPW
        pltpu.sync_copy(feat_hbm.at[pl.ds(base, _RPW)], feat_v)
        pltpu.sync_copy(lab_hbm.at[pl.ds(base, _RPW)], lab_v)
        pltpu.sync_copy(c0_hbm, c0_v)
        pltpu.sync_copy(c1_hbm, c1_v)

        c0 = [c0_v[0, pl.ds(j * _L, _L)] for j in range(_CHUNKS)]
        dlt = [c1_v[0, pl.ds(j * _L, _L)] - c0[j] for j in range(_CHUNKS)]

        def group_body(g, acc):
            base_r = g * _L
            lvf = lab_v[pl.ds(base_r, _L)].astype(jnp.float32)
            for k in range(_L):
                lf = lvf[k]
                for j in range(_CHUNKS):
                    t = (feat_v[base_r + k, pl.ds(j * _L, _L)]
                         - c0[j] - lf * dlt[j])
                    acc = acc + t * t
            return acc

        acc = lax.fori_loop(0, _RPW // _L, group_body,
                            jnp.zeros((_L,), jnp.float32))
        acc_v[...] = acc
        pltpu.sync_copy(acc_v, out_hbm.at[wid])

    return sc_partials


_sc_partials = _make_sc_partials()


def kernel(features, labels, proto_0, proto_1):
    partials = _sc_partials(features, labels.astype(jnp.int32),
                            proto_0, proto_1)
    scale = LAMBDA * 0.5 / features.shape[0]
    return scale * jnp.sum(partials)


# speedup vs baseline: 2.0266x; 1.0045x over previous
import functools

import jax
import jax.numpy as jnp
from jax import lax
from jax.experimental import pallas as pl
from jax.experimental.pallas import tpu as pltpu
from jax.experimental.pallas import tpu_sc as plsc

LAMBDA = 1.0

_NC = 2
_NS = 16
_NW = _NC * _NS
_L = 16

_ROWS = 16384
_D = 128
_RPW = _ROWS // _NW
_CR = 128
_CHUNKS = _D // _L


def _make_sc_partials():
    mesh = plsc.VectorSubcoreMesh(core_axis_name="c", subcore_axis_name="s")

    @functools.partial(
        pl.kernel,
        mesh=mesh,
        out_type=jax.ShapeDtypeStruct((_NW, _L), jnp.float32),
        scratch_types=[
            pltpu.VMEM((_CR, _D), jnp.float32),
            pltpu.VMEM((_RPW,), jnp.int32),
            pltpu.VMEM((_RPW, _L), jnp.float32),
            pltpu.VMEM((1, _D), jnp.float32),
            pltpu.VMEM((1, _D), jnp.float32),
            pltpu.VMEM((_L,), jnp.float32),
        ],
    )
    def sc_partials(feat_hbm, lab_hbm, c0_hbm, c1_hbm, out_hbm,
                    feat_v, lab_v, lab16_v, c0_v, c1_v, acc_v):
        wid = lax.axis_index("s") * _NC + lax.axis_index("c")
        base = wid * _RPW
        pltpu.sync_copy(lab_hbm.at[pl.ds(base, _RPW)], lab_v)
        pltpu.sync_copy(c0_hbm, c0_v)
        pltpu.sync_copy(c1_hbm, c1_v)

        c0 = [c0_v[0, pl.ds(j * _L, _L)] for j in range(_CHUNKS)]
        dlt = [c1_v[0, pl.ds(j * _L, _L)] - c0[j] for j in range(_CHUNKS)]

        def expand_body(g, _):
            lvf = lab_v[pl.ds(g * _L, _L)].astype(jnp.float32)
            for k in range(_L):
                lab16_v[g * _L + k, :] = jnp.full((_L,), lvf[k],
                                                  jnp.float32)
            return 0

        lax.fori_loop(0, _RPW // _L, expand_body, 0)

        def chunk_body(ci, acc):
            pltpu.sync_copy(feat_hbm.at[pl.ds(base + ci * _CR, _CR)],
                            feat_v)

            def row_body(r, acc):
                lf = lab16_v[ci * _CR + r, :]
                for j in range(_CHUNKS):
                    t = (feat_v[r, pl.ds(j * _L, _L)]
                         - c0[j] - lf * dlt[j])
                    acc = acc + t * t
                return acc

            return lax.fori_loop(0, _CR, row_body, acc)

        acc = lax.fori_loop(0, _RPW // _CR, chunk_body,
                            jnp.zeros((_L,), jnp.float32))
        acc_v[...] = acc * (LAMBDA * 0.5 / _ROWS)
        pltpu.sync_copy(acc_v, out_hbm.at[wid])

    return sc_partials


_sc_partials = _make_sc_partials()


def kernel(features, labels, proto_0, proto_1):
    partials = _sc_partials(features, labels.astype(jnp.int32),
                            proto_0, proto_1)
    return jnp.sum(partials)
